# initial kernel scaffold (unmeasured)
import jax
import jax.numpy as jnp
from jax import lax
from jax.experimental import pallas as pl
from jax.experimental.pallas import tpu as pltpu

N_DEV = 32


def kernel(x, w_mat):
    m, k_per = x.shape
    _, n = w_mat.shape
    m_per = m // N_DEV

    def body(x_ref, w_ref, out_ref, partial, recv_buf, send_sems, recv_sems):
        me = lax.axis_index("i")

        p = jnp.dot(x_ref[:, :], w_ref[:, :],
                    preferred_element_type=jnp.float32)
        partial[:, :, :] = p.reshape(N_DEV, m_per, n)

        rdmas = []
        for s in range(1, N_DEV):
            tgt = (me + s) % N_DEV
            rdma = pltpu.make_async_remote_copy(
                src_ref=partial.at[tgt],
                dst_ref=recv_buf.at[s - 1],
                send_sem=send_sems.at[s - 1],
                recv_sem=recv_sems.at[s - 1],
                device_id=(tgt,),
                device_id_type=pl.DeviceIdType.MESH,
            )
            rdma.start()
            rdmas.append(rdma)

        acc = partial[me]
        for s in range(1, N_DEV):
            rdmas[s - 1].wait_recv()
            acc = acc + recv_buf[s - 1]

        c = 0.7978845608028654
        out_ref[:, :] = 0.5 * acc * (
            1.0 + jnp.tanh(c * (acc + 0.044715 * acc * acc * acc))
        )

        for rdma in rdmas:
            rdma.wait_send()

    return pl.pallas_call(
        body,
        out_shape=jax.ShapeDtypeStruct((m_per, n), jnp.float32),
        in_specs=[
            pl.BlockSpec(memory_space=pltpu.VMEM),
            pl.BlockSpec(memory_space=pltpu.VMEM),
        ],
        out_specs=pl.BlockSpec(memory_space=pltpu.VMEM),
        scratch_shapes=[
            pltpu.VMEM((N_DEV, m_per, n), jnp.float32),
            pltpu.VMEM((N_DEV - 1, m_per, n), jnp.float32),
            pltpu.SemaphoreType.DMA((N_DEV - 1,)),
            pltpu.SemaphoreType.DMA((N_DEV - 1,)),
        ],
        compiler_params=pltpu.CompilerParams(collective_id=0),
    )(x, w_mat)


# baseline (device time: 65141 ns/iter reference)
import jax
import jax.numpy as jnp
from jax import lax
from jax.experimental import pallas as pl
from jax.experimental.pallas import tpu as pltpu

N_DEV = 32


def kernel(x, w_mat):
    m, k_per = x.shape
    _, n = w_mat.shape
    m_per = m // N_DEV

    def body(x_ref, w_ref, out_ref, partial, recv_buf, send_sems, recv_sems):
        me = lax.axis_index("i")

        p = jnp.dot(x_ref[:, :], w_ref[:, :],
                    preferred_element_type=jnp.float32)
        partial[:, :, :] = p.reshape(N_DEV, m_per, n)

        rdmas = []
        for s in range(1, N_DEV):
            tgt = (me + s) % N_DEV
            rdma = pltpu.make_async_remote_copy(
                src_ref=partial.at[tgt],
                dst_ref=recv_buf.at[s - 1],
                send_sem=send_sems.at[s - 1],
                recv_sem=recv_sems.at[s - 1],
                device_id=(tgt,),
                device_id_type=pl.DeviceIdType.MESH,
            )
            rdma.start()
            rdmas.append(rdma)

        acc = partial[me]
        for s in range(1, N_DEV):
            rdmas[s - 1].wait_recv()
            acc = acc + recv_buf[s - 1]

        c = 0.7978845608028654
        out_ref[:, :] = 0.5 * acc * (
            1.0 + jnp.tanh(c * (acc + 0.044715 * acc * acc * acc))
        )

        for rdma in rdmas:
            rdma.wait_send()

    return pl.pallas_call(
        body,
        out_shape=jax.ShapeDtypeStruct((m_per, n), jnp.float32),
        in_specs=[
            pl.BlockSpec(memory_space=pltpu.VMEM),
            pl.BlockSpec(memory_space=pltpu.VMEM),
        ],
        out_specs=pl.BlockSpec(memory_space=pltpu.VMEM),
        scratch_shapes=[
            pltpu.VMEM((N_DEV, m_per, n), jnp.float32),
            pltpu.VMEM((N_DEV - 1, m_per, n), jnp.float32),
            pltpu.SemaphoreType.DMA((N_DEV - 1,)),
            pltpu.SemaphoreType.DMA((N_DEV - 1,)),
        ],
    )(x, w_mat)


# device time: 38932 ns/iter; 1.6732x vs baseline; 1.6732x over previous
import jax
import jax.numpy as jnp
from jax import lax
from jax.experimental import pallas as pl
from jax.experimental.pallas import tpu as pltpu

N_DEV = 32


def kernel(x, w_mat):
    m, k_per = x.shape
    _, n = w_mat.shape
    m_per = m // N_DEV

    def body(x_ref, w_ref, out_ref, partial, recv_buf, send_sems, recv_sems):
        me = lax.axis_index("i")

        p = jnp.dot(x_ref[:, :], w_ref[:, :],
                    preferred_element_type=jnp.float32)
        partial[:, :, :] = p.reshape(N_DEV, m_per, n).astype(jnp.bfloat16)

        rdmas = []
        for s in range(1, N_DEV):
            tgt = (me + s) % N_DEV
            rdma = pltpu.make_async_remote_copy(
                src_ref=partial.at[tgt],
                dst_ref=recv_buf.at[s - 1],
                send_sem=send_sems.at[s - 1],
                recv_sem=recv_sems.at[s - 1],
                device_id=(tgt,),
                device_id_type=pl.DeviceIdType.MESH,
            )
            rdma.start()
            rdmas.append(rdma)

        acc = partial[me].astype(jnp.float32)
        for s in range(1, N_DEV):
            rdmas[s - 1].wait_recv()
            acc = acc + recv_buf[s - 1].astype(jnp.float32)

        c = 0.7978845608028654
        out_ref[:, :] = 0.5 * acc * (
            1.0 + jnp.tanh(c * (acc + 0.044715 * acc * acc * acc))
        )

        for rdma in rdmas:
            rdma.wait_send()

    return pl.pallas_call(
        body,
        out_shape=jax.ShapeDtypeStruct((m_per, n), jnp.float32),
        in_specs=[
            pl.BlockSpec(memory_space=pltpu.VMEM),
            pl.BlockSpec(memory_space=pltpu.VMEM),
        ],
        out_specs=pl.BlockSpec(memory_space=pltpu.VMEM),
        scratch_shapes=[
            pltpu.VMEM((N_DEV, m_per, n), jnp.bfloat16),
            pltpu.VMEM((N_DEV - 1, m_per, n), jnp.bfloat16),
            pltpu.SemaphoreType.DMA((N_DEV - 1,)),
            pltpu.SemaphoreType.DMA((N_DEV - 1,)),
        ],
    )(x, w_mat)


# device time: 35553 ns/iter; 1.8322x vs baseline; 1.0950x over previous
import jax
import jax.numpy as jnp
from jax import lax
from jax.experimental import pallas as pl
from jax.experimental.pallas import tpu as pltpu

N_DEV = 32
NX, NY, NZ = 2, 4, 4

_PERM8 = tuple(
    yp * 2 + (xg if yp % 2 == 0 else 1 - xg)
    for xg in range(NX)
    for yp in range(NY)
)


def kernel(x, w_mat):
    m, k_per = x.shape
    _, n = w_mat.shape
    m_per = m // N_DEV
    m_blk = m // NZ

    def body(x_ref, w_ref, out_ref,
             partial, xrecv, axbuf, yrecv, bzbuf, zrecv,
             xsend_sems, xrecv_sems, ysend_sems, yrecv_sems,
             zsend_sems, zrecv_sems):
        me = lax.axis_index("i")
        z = me // 8
        r = me % 8
        yy = r // 2
        j = r % 2
        xx = jnp.where(yy % 2 == 0, j, 1 - j)

        qs = [(z + 1 + kq) % NZ if kq < NZ - 1 else z for kq in range(NZ)]

        x_rdmas = []
        for kq in range(NZ):
            q = qs[kq]
            p_q = jnp.dot(
                x_ref[pl.ds(q * m_blk, m_blk), :], w_ref[:, :],
                preferred_element_type=jnp.float32,
            )
            chunks = p_q.astype(jnp.bfloat16).reshape(NX * NY, m_per, n)
            partial[kq] = jnp.stack(
                [chunks[t] for t in _PERM8]
            ).reshape(NX, NY, m_per, n)

            rd = pltpu.make_async_remote_copy(
                src_ref=partial.at[kq, 1 - xx],
                dst_ref=xrecv.at[kq],
                send_sem=xsend_sems.at[kq],
                recv_sem=xrecv_sems.at[kq],
                device_id=(me ^ 1,),
                device_id_type=pl.DeviceIdType.MESH,
            )
            rd.start()
            x_rdmas.append(rd)

        y_rdmas = []
        z_rdmas = []
        final_acc = None
        for kq in range(NZ):
            q = qs[kq]
            x_rdmas[kq].wait_recv()
            a_q = (partial[kq, xx].astype(jnp.float32)
                   + xrecv[kq].astype(jnp.float32))
            axbuf[kq] = a_q.astype(jnp.bfloat16)

            yr_block = []
            for k in range(NY - 1):
                yp = (yy + 1 + k) % NY
                jp = jnp.where(yp % 2 == 0, xx, 1 - xx)
                tgt = z * 8 + yp * 2 + jp
                rd = pltpu.make_async_remote_copy(
                    src_ref=axbuf.at[kq, yp],
                    dst_ref=yrecv.at[kq, k],
                    send_sem=ysend_sems.at[kq, k],
                    recv_sem=yrecv_sems.at[kq, k],
                    device_id=(tgt,),
                    device_id_type=pl.DeviceIdType.MESH,
                )
                rd.start()
                yr_block.append(rd)
            y_rdmas.extend(yr_block)

            acc_q = (partial[kq, xx, yy].astype(jnp.float32)
                     + xrecv[kq, yy].astype(jnp.float32))
            for k in range(NY - 1):
                yr_block[k].wait_recv()
                acc_q = acc_q + yrecv[kq, k].astype(jnp.float32)

            if kq < NZ - 1:
                bzbuf[kq] = acc_q.astype(jnp.bfloat16)
                rd = pltpu.make_async_remote_copy(
                    src_ref=bzbuf.at[kq],
                    dst_ref=zrecv.at[kq],
                    send_sem=zsend_sems.at[kq],
                    recv_sem=zrecv_sems.at[kq],
                    device_id=(q * 8 + r,),
                    device_id_type=pl.DeviceIdType.MESH,
                )
                rd.start()
                z_rdmas.append(rd)
            else:
                final_acc = acc_q

        for k in range(NZ - 1):
            z_rdmas[k].wait_recv()
            final_acc = final_acc + zrecv[k].astype(jnp.float32)

        c = 0.7978845608028654
        out_ref[:, :] = 0.5 * final_acc * (
            1.0 + jnp.tanh(c * (final_acc
                                + 0.044715 * final_acc * final_acc * final_acc))
        )

        for rd in x_rdmas + y_rdmas + z_rdmas:
            rd.wait_send()

    return pl.pallas_call(
        body,
        out_shape=jax.ShapeDtypeStruct((m_per, n), jnp.float32),
        in_specs=[
            pl.BlockSpec(memory_space=pltpu.VMEM),
            pl.BlockSpec(memory_space=pltpu.VMEM),
        ],
        out_specs=pl.BlockSpec(memory_space=pltpu.VMEM),
        scratch_shapes=[
            pltpu.VMEM((NZ, NX, NY, m_per, n), jnp.bfloat16),
            pltpu.VMEM((NZ, NY, m_per, n), jnp.bfloat16),
            pltpu.VMEM((NZ, NY, m_per, n), jnp.bfloat16),
            pltpu.VMEM((NZ, NY - 1, m_per, n), jnp.bfloat16),
            pltpu.VMEM((NZ - 1, m_per, n), jnp.bfloat16),
            pltpu.VMEM((NZ - 1, m_per, n), jnp.bfloat16),
            pltpu.SemaphoreType.DMA((NZ,)),
            pltpu.SemaphoreType.DMA((NZ,)),
            pltpu.SemaphoreType.DMA((NZ, NY - 1)),
            pltpu.SemaphoreType.DMA((NZ, NY - 1)),
            pltpu.SemaphoreType.DMA((NZ - 1,)),
            pltpu.SemaphoreType.DMA((NZ - 1,)),
        ],
    )(x, w_mat)


# device time: 30622 ns/iter; 2.1273x vs baseline; 1.1610x over previous
import jax
import jax.numpy as jnp
from jax import lax
from jax.experimental import pallas as pl
from jax.experimental.pallas import tpu as pltpu

N_DEV = 32
NX, NY, NZ = 2, 4, 4

_PERM8 = tuple(
    yp * 2 + (xg if yp % 2 == 0 else 1 - xg)
    for xg in range(NX)
    for yp in range(NY)
)


def kernel(x, w_mat):
    m, k_per = x.shape
    _, n = w_mat.shape
    m_per = m // N_DEV
    m_blk = m // NZ

    def body(x_ref, w_ref, out_ref,
             partial, xrecv, axbuf, yrecv, bzbuf, zrecv,
             xsend_sems, xrecv_sems, ysend_sems, yrecv_sems,
             zsend_sems, zrecv_sems):
        me = lax.axis_index("i")
        z = me // 8
        r = me % 8
        yy = r // 2
        j = r % 2
        xx = jnp.where(yy % 2 == 0, j, 1 - j)

        qs = [(z + 1 + kq) % NZ if kq < NZ - 1 else z for kq in range(NZ)]

        barrier_sem = pltpu.get_barrier_semaphore()
        pl.semaphore_signal(
            barrier_sem, inc=1,
            device_id=(me ^ 1,), device_id_type=pl.DeviceIdType.MESH,
        )
        for k in range(NY - 1):
            yp = (yy + 1 + k) % NY
            jp = jnp.where(yp % 2 == 0, xx, 1 - xx)
            pl.semaphore_signal(
                barrier_sem, inc=1,
                device_id=(z * 8 + yp * 2 + jp,),
                device_id_type=pl.DeviceIdType.MESH,
            )
        for k in range(NZ - 1):
            zp = (z + 1 + k) % NZ
            pl.semaphore_signal(
                barrier_sem, inc=1,
                device_id=(zp * 8 + r,),
                device_id_type=pl.DeviceIdType.MESH,
            )
        pl.semaphore_wait(barrier_sem, 7)

        x_rdmas = []
        for kq in range(NZ):
            q = qs[kq]
            p_q = jnp.dot(
                x_ref[pl.ds(q * m_blk, m_blk), :], w_ref[:, :],
                preferred_element_type=jnp.float32,
            )
            chunks = p_q.astype(jnp.bfloat16).reshape(NX * NY, m_per, n)
            partial[kq] = jnp.stack(
                [chunks[t] for t in _PERM8]
            ).reshape(NX, NY, m_per, n)

            rd = pltpu.make_async_remote_copy(
                src_ref=partial.at[kq, 1 - xx],
                dst_ref=xrecv.at[kq],
                send_sem=xsend_sems.at[kq],
                recv_sem=xrecv_sems.at[kq],
                device_id=(me ^ 1,),
                device_id_type=pl.DeviceIdType.MESH,
            )
            rd.start()
            x_rdmas.append(rd)

        y_rdmas = []
        z_rdmas = []
        final_acc = None
        for kq in range(NZ):
            q = qs[kq]
            x_rdmas[kq].wait_recv()
            a_q = (partial[kq, xx].astype(jnp.float32)
                   + xrecv[kq].astype(jnp.float32))
            axbuf[kq] = a_q.astype(jnp.bfloat16)

            yr_block = []
            for k in range(NY - 1):
                yp = (yy + 1 + k) % NY
                jp = jnp.where(yp % 2 == 0, xx, 1 - xx)
                tgt = z * 8 + yp * 2 + jp
                rd = pltpu.make_async_remote_copy(
                    src_ref=axbuf.at[kq, yp],
                    dst_ref=yrecv.at[kq, k],
                    send_sem=ysend_sems.at[kq, k],
                    recv_sem=yrecv_sems.at[kq, k],
                    device_id=(tgt,),
                    device_id_type=pl.DeviceIdType.MESH,
                )
                rd.start()
                yr_block.append(rd)
            y_rdmas.extend(yr_block)

            acc_q = (partial[kq, xx, yy].astype(jnp.float32)
                     + xrecv[kq, yy].astype(jnp.float32))
            for k in range(NY - 1):
                yr_block[k].wait_recv()
                acc_q = acc_q + yrecv[kq, k].astype(jnp.float32)

            if kq < NZ - 1:
                bzbuf[kq] = acc_q.astype(jnp.bfloat16)
                rd = pltpu.make_async_remote_copy(
                    src_ref=bzbuf.at[kq],
                    dst_ref=zrecv.at[kq],
                    send_sem=zsend_sems.at[kq],
                    recv_sem=zrecv_sems.at[kq],
                    device_id=(q * 8 + r,),
                    device_id_type=pl.DeviceIdType.MESH,
                )
                rd.start()
                z_rdmas.append(rd)
            else:
                final_acc = acc_q

        for k in range(NZ - 1):
            z_rdmas[k].wait_recv()
            final_acc = final_acc + zrecv[k].astype(jnp.float32)

        c = 0.7978845608028654
        out_ref[:, :] = 0.5 * final_acc * (
            1.0 + jnp.tanh(c * (final_acc
                                + 0.044715 * final_acc * final_acc * final_acc))
        )

        for rd in x_rdmas + y_rdmas + z_rdmas:
            rd.wait_send()

    return pl.pallas_call(
        body,
        out_shape=jax.ShapeDtypeStruct((m_per, n), jnp.float32),
        in_specs=[
            pl.BlockSpec(memory_space=pltpu.VMEM),
            pl.BlockSpec(memory_space=pltpu.VMEM),
        ],
        out_specs=pl.BlockSpec(memory_space=pltpu.VMEM),
        scratch_shapes=[
            pltpu.VMEM((NZ, NX, NY, m_per, n), jnp.bfloat16),
            pltpu.VMEM((NZ, NY, m_per, n), jnp.bfloat16),
            pltpu.VMEM((NZ, NY, m_per, n), jnp.bfloat16),
            pltpu.VMEM((NZ, NY - 1, m_per, n), jnp.bfloat16),
            pltpu.VMEM((NZ - 1, m_per, n), jnp.bfloat16),
            pltpu.VMEM((NZ - 1, m_per, n), jnp.bfloat16),
            pltpu.SemaphoreType.DMA((NZ,)),
            pltpu.SemaphoreType.DMA((NZ,)),
            pltpu.SemaphoreType.DMA((NZ, NY - 1)),
            pltpu.SemaphoreType.DMA((NZ, NY - 1)),
            pltpu.SemaphoreType.DMA((NZ - 1,)),
            pltpu.SemaphoreType.DMA((NZ - 1,)),
        ],
        compiler_params=pltpu.CompilerParams(collective_id=0),
    )(x, w_mat)


# device time: 25619 ns/iter; 2.5427x vs baseline; 1.1953x over previous
import jax
import jax.numpy as jnp
from jax import lax
from jax.experimental import pallas as pl
from jax.experimental.pallas import tpu as pltpu

N_DEV = 32
NX, NY, NZ = 2, 4, 4

_PERM8 = tuple(
    yp * 2 + (xg if yp % 2 == 0 else 1 - xg)
    for xg in range(NX)
    for yp in range(NY)
)


def kernel(x, w_mat):
    m, k_per = x.shape
    _, n = w_mat.shape
    m_per = m // N_DEV
    m_blk = m // NZ

    def body(x_ref, w_ref, out_ref,
             partial, xrecv, axbuf, yrecv, bzbuf, zrecv,
             xsend_sems, xrecv_sems, ysend_sems, yrecv_sems,
             zsend_sems, zrecv_sems):
        me = lax.axis_index("i")
        z = me // 8
        r = me % 8
        yy = r // 2
        j = r % 2
        xx = jnp.where(yy % 2 == 0, j, 1 - j)

        qs = [(z + 1 + kq) % NZ if kq < NZ - 1 else z for kq in range(NZ)]

        barrier_sem = pltpu.get_barrier_semaphore()
        pl.semaphore_signal(
            barrier_sem, inc=1,
            device_id=(me ^ 1,), device_id_type=pl.DeviceIdType.MESH,
        )
        for k in range(NY - 1):
            yp = (yy + 1 + k) % NY
            jp = jnp.where(yp % 2 == 0, xx, 1 - xx)
            pl.semaphore_signal(
                barrier_sem, inc=1,
                device_id=(z * 8 + yp * 2 + jp,),
                device_id_type=pl.DeviceIdType.MESH,
            )
        for k in range(NZ - 1):
            zp = (z + 1 + k) % NZ
            pl.semaphore_signal(
                barrier_sem, inc=1,
                device_id=(zp * 8 + r,),
                device_id_type=pl.DeviceIdType.MESH,
            )

        x_rdmas = []
        for kq in range(NZ):
            q = qs[kq]
            p_q = jnp.dot(
                x_ref[pl.ds(q * m_blk, m_blk), :], w_ref[:, :],
                preferred_element_type=jnp.float32,
            )
            chunks = p_q.astype(jnp.bfloat16).reshape(NX * NY, m_per, n)
            partial[kq] = jnp.stack(
                [chunks[t] for t in _PERM8]
            ).reshape(NX, NY, m_per, n)

            if kq == 0:
                pl.semaphore_wait(barrier_sem, 7)
            rd = pltpu.make_async_remote_copy(
                src_ref=partial.at[kq, 1 - xx],
                dst_ref=xrecv.at[kq],
                send_sem=xsend_sems.at[kq],
                recv_sem=xrecv_sems.at[kq],
                device_id=(me ^ 1,),
                device_id_type=pl.DeviceIdType.MESH,
            )
            rd.start()
            x_rdmas.append(rd)

        y_rdmas = []
        for kq in range(NZ):
            x_rdmas[kq].wait_recv()
            a_q = (partial[kq, xx].astype(jnp.float32)
                   + xrecv[kq].astype(jnp.float32))
            axbuf[kq] = a_q.astype(jnp.bfloat16)

            for k in range(NY - 1):
                yp = (yy + 1 + k) % NY
                jp = jnp.where(yp % 2 == 0, xx, 1 - xx)
                tgt = z * 8 + yp * 2 + jp
                rd = pltpu.make_async_remote_copy(
                    src_ref=axbuf.at[kq, yp],
                    dst_ref=yrecv.at[kq, k],
                    send_sem=ysend_sems.at[kq, k],
                    recv_sem=yrecv_sems.at[kq, k],
                    device_id=(tgt,),
                    device_id_type=pl.DeviceIdType.MESH,
                )
                rd.start()
                y_rdmas.append(rd)

        z_rdmas = []
        final_acc = None
        for kq in range(NZ):
            q = qs[kq]
            acc_q = (partial[kq, xx, yy].astype(jnp.float32)
                     + xrecv[kq, yy].astype(jnp.float32))
            for k in range(NY - 1):
                y_rdmas[kq * (NY - 1) + k].wait_recv()
                acc_q = acc_q + yrecv[kq, k].astype(jnp.float32)

            if kq < NZ - 1:
                bzbuf[kq] = acc_q.astype(jnp.bfloat16)
                rd = pltpu.make_async_remote_copy(
                    src_ref=bzbuf.at[kq],
                    dst_ref=zrecv.at[kq],
                    send_sem=zsend_sems.at[kq],
                    recv_sem=zrecv_sems.at[kq],
                    device_id=(q * 8 + r,),
                    device_id_type=pl.DeviceIdType.MESH,
                )
                rd.start()
                z_rdmas.append(rd)
            else:
                final_acc = acc_q

        for k in range(NZ - 1):
            z_rdmas[k].wait_recv()
            final_acc = final_acc + zrecv[k].astype(jnp.float32)

        c = 0.7978845608028654
        out_ref[:, :] = 0.5 * final_acc * (
            1.0 + jnp.tanh(c * (final_acc
                                + 0.044715 * final_acc * final_acc * final_acc))
        )

        for rd in x_rdmas + y_rdmas + z_rdmas:
            rd.wait_send()

    return pl.pallas_call(
        body,
        out_shape=jax.ShapeDtypeStruct((m_per, n), jnp.float32),
        in_specs=[
            pl.BlockSpec(memory_space=pltpu.VMEM),
            pl.BlockSpec(memory_space=pltpu.VMEM),
        ],
        out_specs=pl.BlockSpec(memory_space=pltpu.VMEM),
        scratch_shapes=[
            pltpu.VMEM((NZ, NX, NY, m_per, n), jnp.bfloat16),
            pltpu.VMEM((NZ, NY, m_per, n), jnp.bfloat16),
            pltpu.VMEM((NZ, NY, m_per, n), jnp.bfloat16),
            pltpu.VMEM((NZ, NY - 1, m_per, n), jnp.bfloat16),
            pltpu.VMEM((NZ - 1, m_per, n), jnp.bfloat16),
            pltpu.VMEM((NZ - 1, m_per, n), jnp.bfloat16),
            pltpu.SemaphoreType.DMA((NZ,)),
            pltpu.SemaphoreType.DMA((NZ,)),
            pltpu.SemaphoreType.DMA((NZ, NY - 1)),
            pltpu.SemaphoreType.DMA((NZ, NY - 1)),
            pltpu.SemaphoreType.DMA((NZ - 1,)),
            pltpu.SemaphoreType.DMA((NZ - 1,)),
        ],
        compiler_params=pltpu.CompilerParams(collective_id=0),
    )(x, w_mat)


# device time: 25600 ns/iter; 2.5446x vs baseline; 1.0007x over previous
import jax
import jax.numpy as jnp
from jax import lax
from jax.experimental import pallas as pl
from jax.experimental.pallas import tpu as pltpu

N_DEV = 32
NX, NY, NZ = 2, 4, 4

_PERM8 = tuple(
    yp * 2 + (xg if yp % 2 == 0 else 1 - xg)
    for xg in range(NX)
    for yp in range(NY)
)


def kernel(x, w_mat):
    m, k_per = x.shape
    _, n = w_mat.shape
    m_per = m // N_DEV
    m_blk = m // NZ

    def body(x_ref, w_ref, out_ref,
             partial, xrecv, axbuf, yrecv, bzbuf, zrecv,
             xsend_sems, xrecv_sems, ysend_sems, yrecv_sems,
             zsend_sems, zrecv_sems):
        me = lax.axis_index("i")
        z = me // 8
        r = me % 8
        yy = r // 2
        j = r % 2
        xx = jnp.where(yy % 2 == 0, j, 1 - j)

        qs = [(z + 1 + kq) % NZ if kq < NZ - 1 else z for kq in range(NZ)]

        barrier_sem = pltpu.get_barrier_semaphore()
        pl.semaphore_signal(
            barrier_sem, inc=1,
            device_id=(me ^ 1,), device_id_type=pl.DeviceIdType.MESH,
        )
        for k in range(NY - 1):
            yp = (yy + 1 + k) % NY
            jp = jnp.where(yp % 2 == 0, xx, 1 - xx)
            pl.semaphore_signal(
                barrier_sem, inc=1,
                device_id=(z * 8 + yp * 2 + jp,),
                device_id_type=pl.DeviceIdType.MESH,
            )
        for k in range(NZ - 1):
            zp = (z + 1 + k) % NZ
            pl.semaphore_signal(
                barrier_sem, inc=1,
                device_id=(zp * 8 + r,),
                device_id_type=pl.DeviceIdType.MESH,
            )

        x_rdmas = []
        for kq in range(NZ):
            q = qs[kq]
            p_q = jnp.dot(
                x_ref[pl.ds(q * m_blk, m_blk), :], w_ref[:, :],
                preferred_element_type=jnp.float32,
            )
            chunks = p_q.astype(jnp.bfloat16).reshape(NX * NY, m_per, n)
            partial[kq] = jnp.stack(
                [chunks[t] for t in _PERM8]
            ).reshape(NX, NY, m_per, n)

            if kq == 0:
                pl.semaphore_wait(barrier_sem, 7)
            rd = pltpu.make_async_remote_copy(
                src_ref=partial.at[kq, 1 - xx],
                dst_ref=xrecv.at[kq],
                send_sem=xsend_sems.at[kq],
                recv_sem=xrecv_sems.at[kq],
                device_id=(me ^ 1,),
                device_id_type=pl.DeviceIdType.MESH,
            )
            rd.start()
            x_rdmas.append(rd)

        y_rdmas = []
        for kq in range(NZ):
            x_rdmas[kq].wait_recv()
            axbuf[kq] = partial[kq, xx] + xrecv[kq]

            for k in range(NY - 1):
                yp = (yy + 1 + k) % NY
                jp = jnp.where(yp % 2 == 0, xx, 1 - xx)
                tgt = z * 8 + yp * 2 + jp
                rd = pltpu.make_async_remote_copy(
                    src_ref=axbuf.at[kq, yp],
                    dst_ref=yrecv.at[kq, k],
                    send_sem=ysend_sems.at[kq, k],
                    recv_sem=yrecv_sems.at[kq, k],
                    device_id=(tgt,),
                    device_id_type=pl.DeviceIdType.MESH,
                )
                rd.start()
                y_rdmas.append(rd)

        z_rdmas = []
        final_acc = None
        for kq in range(NZ):
            q = qs[kq]
            acc_q = (partial[kq, xx, yy].astype(jnp.float32)
                     + xrecv[kq, yy].astype(jnp.float32))
            for k in range(NY - 1):
                y_rdmas[kq * (NY - 1) + k].wait_recv()
                acc_q = acc_q + yrecv[kq, k].astype(jnp.float32)

            if kq < NZ - 1:
                bzbuf[kq] = acc_q.astype(jnp.bfloat16)
                rd = pltpu.make_async_remote_copy(
                    src_ref=bzbuf.at[kq],
                    dst_ref=zrecv.at[kq],
                    send_sem=zsend_sems.at[kq],
                    recv_sem=zrecv_sems.at[kq],
                    device_id=(q * 8 + r,),
                    device_id_type=pl.DeviceIdType.MESH,
                )
                rd.start()
                z_rdmas.append(rd)
            else:
                final_acc = acc_q

        for k in range(NZ - 1):
            z_rdmas[k].wait_recv()
            final_acc = final_acc + zrecv[k].astype(jnp.float32)

        c = 0.7978845608028654
        out_ref[:, :] = 0.5 * final_acc * (
            1.0 + jnp.tanh(c * (final_acc
                                + 0.044715 * final_acc * final_acc * final_acc))
        )

        for rd in x_rdmas + y_rdmas + z_rdmas:
            rd.wait_send()

    return pl.pallas_call(
        body,
        out_shape=jax.ShapeDtypeStruct((m_per, n), jnp.float32),
        in_specs=[
            pl.BlockSpec(memory_space=pltpu.VMEM),
            pl.BlockSpec(memory_space=pltpu.VMEM),
        ],
        out_specs=pl.BlockSpec(memory_space=pltpu.VMEM),
        scratch_shapes=[
            pltpu.VMEM((NZ, NX, NY, m_per, n), jnp.bfloat16),
            pltpu.VMEM((NZ, NY, m_per, n), jnp.bfloat16),
            pltpu.VMEM((NZ, NY, m_per, n), jnp.bfloat16),
            pltpu.VMEM((NZ, NY - 1, m_per, n), jnp.bfloat16),
            pltpu.VMEM((NZ - 1, m_per, n), jnp.bfloat16),
            pltpu.VMEM((NZ - 1, m_per, n), jnp.bfloat16),
            pltpu.SemaphoreType.DMA((NZ,)),
            pltpu.SemaphoreType.DMA((NZ,)),
            pltpu.SemaphoreType.DMA((NZ, NY - 1)),
            pltpu.SemaphoreType.DMA((NZ, NY - 1)),
            pltpu.SemaphoreType.DMA((NZ - 1,)),
            pltpu.SemaphoreType.DMA((NZ - 1,)),
        ],
        compiler_params=pltpu.CompilerParams(collective_id=0),
    )(x, w_mat)
